# Initial kernel scaffold; baseline (speedup 1.0000x reference)
import jax
import jax.numpy as jnp
from jax.experimental import pallas as pl


def kernel(s0, s1, weights, selected_idxs, pre0_w, pre0_g, pre0_b, pre1_w, pre1_g, pre1_b, conv_w, conv_g, conv_b, edge_w, edge_g, edge_b, mr_w, mr_g, mr_b):
    raise NotImplementedError("write your pallas kernel here")


# speedup vs baseline: 454.2928x; 454.2928x over previous
import functools

import jax
import jax.numpy as jnp
from jax import lax
from jax.experimental import pallas as pl
from jax.experimental.pallas import tpu as pltpu
from jax.experimental.pallas import tpu_sc as plsc

B = 4
C = 64
N = 1024
K = 9
R = B * N
E = R * K
STEPS = 3
MULT = 3
EPS = 1e-5


_NCORES = 2
_NSUB = 16
_NW = _NCORES * _NSUB
_RPW = E // _NW
_CHUNK = 128
_NCH = _RPW // _CHUNK


def _sc_gather_body(table_hbm, gidx_hbm, out_hbm, idx_v, rows_v, sem):
    wid = lax.axis_index("s") * _NCORES + lax.axis_index("c")
    base = wid * _RPW
    pltpu.sync_copy(gidx_hbm.at[pl.ds(base, _RPW)], idx_v)
    copies = [
        pltpu.async_copy(
            table_hbm.at[idx_v.at[pl.ds(c * _CHUNK, _CHUNK)]],
            rows_v.at[pl.ds(c * _CHUNK, _CHUNK)],
            sem,
        )
        for c in range(_NCH)
    ]
    for cp in copies:
        cp.wait()
    pltpu.sync_copy(rows_v, out_hbm.at[pl.ds(base, _RPW)])


def _sc_gather(table, gidx):
    mesh = plsc.VectorSubcoreMesh(core_axis_name="c", subcore_axis_name="s")
    return pl.kernel(
        _sc_gather_body,
        out_type=jax.ShapeDtypeStruct((E, C), jnp.float32),
        mesh=mesh,
        scratch_types=[
            pltpu.VMEM((_RPW,), jnp.int32),
            pltpu.VMEM((_RPW, C), jnp.float32),
            pltpu.SemaphoreType.DMA,
        ],
        compiler_params=pltpu.CompilerParams(use_tc_tiling_on_sc=False),
    )(table, gidx)


_RB = 32
_CB = 512
_NCHUNK = R // _CB


def _matT(x, w):
    return lax.dot_general(x, w, (((1,), (1,)), ((), ())),
                           preferred_element_type=jnp.float32)


def _bn_prep(s, q, cnt):
    mu = s / cnt
    var = q / cnt - mu * mu
    inv = lax.rsqrt(var + EPS)
    return mu, inv


def _k1_body(s0_ref, s1_ref, p0w_ref, p0g_ref, p0b_ref,
             p1w_ref, p1g_ref, p1b_ref,
             gidx_ref, s0p_ref, s1p_ref):
    col_iota = lax.broadcasted_iota(jnp.int32, (_RB, N), 1)

    for b in range(B):
        x_full = s0_ref[pl.ds(b * N, N), :]
        sq_full = jnp.sum(x_full * x_full, axis=1)
        for blk in range(N // _RB):
            r0 = blk * _RB
            x_blk = s0_ref[pl.ds(b * N + r0, _RB), :]
            sq_blk = jnp.sum(x_blk * x_blk, axis=1)
            d = (sq_blk[:, None]
                 - 2.0 * lax.dot_general(x_blk, x_full,
                                         (((1,), (1,)), ((), ())),
                                         preferred_element_type=jnp.float32)
                 + sq_full[None, :])
            for j in range(K):
                rmin = jnp.min(d, axis=1)
                amin = jnp.min(
                    jnp.where(d == rmin[:, None], col_iota, N), axis=1)
                d = jnp.where(col_iota == amin[:, None], jnp.float32(1e30), d)
                gidx_ref[j, pl.ds(b * N + r0, _RB)] = amin + b * N

    for src_ref, w_ref, g_ref, b_ref, dst_ref in (
            (s0_ref, p0w_ref, p0g_ref, p0b_ref, s0p_ref),
            (s1_ref, p1w_ref, p1g_ref, p1b_ref, s1p_ref)):
        w = w_ref[...]
        acc_s = jnp.zeros((C,), jnp.float32)
        acc_q = jnp.zeros((C,), jnp.float32)
        for ch in range(_NCHUNK):
            y = _matT(src_ref[pl.ds(ch * _CB, _CB), :], w)
            dst_ref[pl.ds(ch * _CB, _CB), :] = y
            acc_s = acc_s + jnp.sum(y, axis=0)
            acc_q = acc_q + jnp.sum(y * y, axis=0)
        mu, inv = _bn_prep(acc_s, acc_q, jnp.float32(R))
        g = g_ref[...]
        bb = b_ref[...]
        for ch in range(_NCHUNK):
            y = dst_ref[pl.ds(ch * _CB, _CB), :]
            dst_ref[pl.ds(ch * _CB, _CB), :] = jax.nn.relu(
                (y - mu[None, :]) * (inv * g)[None, :] + bb[None, :])


def _k1(s0_rows, s1_rows, p0w, p0g, p0b, p1w, p1g, p1b):
    return pl.pallas_call(
        _k1_body,
        out_shape=(
            jax.ShapeDtypeStruct((K, R), jnp.int32),
            jax.ShapeDtypeStruct((R, C), jnp.float32),
            jax.ShapeDtypeStruct((R, C), jnp.float32),
        ),
    )(s0_rows, s1_rows, p0w, p0g, p0b, p1w, p1g, p1b)


def _opgroup_body(op_state, n_states, has_base, refs):
    it = iter(refs)
    h_ref = next(it)
    g_ref = next(it)
    cw_ref, cg_ref, cb_ref = next(it), next(it), next(it)
    ew_ref, eg_ref, eb_ref = next(it), next(it), next(it)
    mw_ref, mg_ref, mb_ref = next(it), next(it), next(it)
    weff_ref = next(it)
    base_ref = next(it) if has_base else None
    out_ref = next(it)
    convmr_ref, yext_ref, hmax_ref = next(it), next(it), next(it)

    n_ops = len(op_state)

    for ch in range(_NCHUNK):
        sl = pl.ds(ch * _CB, _CB)
        m = g_ref[0, sl, :]
        for j in range(1, K):
            m = jnp.maximum(m, g_ref[j, sl, :])
        hmax_ref[sl, :] = m

    for oi in range(n_ops):
        si = op_state[oi]
        w_skip = weff_ref[oi, 1]
        w_conv = weff_ref[oi, 2]
        w_edge = weff_ref[oi, 3]
        w_mr = weff_ref[oi, 4]

        cw = cw_ref[oi]
        ew_r = ew_ref[oi, :, C:]
        ew_d = ew_ref[oi, :, :C] - ew_r
        mw_r = mw_ref[oi, :, C:]
        mw_d = mw_ref[oi, :, :C] - mw_r

        cs = jnp.zeros((C,), jnp.float32)
        cq = jnp.zeros((C,), jnp.float32)
        es = jnp.zeros((C,), jnp.float32)
        eq = jnp.zeros((C,), jnp.float32)
        ms = jnp.zeros((C,), jnp.float32)
        mq = jnp.zeros((C,), jnp.float32)

        for ch in range(_NCHUNK):
            sl = pl.ds(ch * _CB, _CB)
            h = h_ref[sl, si * C:(si + 1) * C]
            hm = hmax_ref[sl, si * C:(si + 1) * C]

            cy = _matT(h, cw)
            cs = cs + jnp.sum(cy, axis=0)
            cq = cq + jnp.sum(cy * cy, axis=0)

            u = _matT(h, ew_d)
            ym = None
            yn = None
            for j in range(K):
                yj = _matT(g_ref[j, sl, si * C:(si + 1) * C], ew_r) + u
                es = es + jnp.sum(yj, axis=0)
                eq = eq + jnp.sum(yj * yj, axis=0)
                ym = yj if ym is None else jnp.maximum(ym, yj)
                yn = yj if yn is None else jnp.minimum(yn, yj)
            yext_ref[sl, :] = jnp.concatenate([ym, yn], axis=1)

            my = _matT(h, mw_d) + _matT(hm, mw_r)
            convmr_ref[sl, :] = jnp.concatenate([cy, my], axis=1)
            ms = ms + jnp.sum(my, axis=0)
            mq = mq + jnp.sum(my * my, axis=0)

        cmu, cinv = _bn_prep(cs, cq, jnp.float32(R))
        emu, einv = _bn_prep(es, eq, jnp.float32(R * K))
        mmu, minv = _bn_prep(ms, mq, jnp.float32(R))

        cg = cg_ref[oi]
        cb = cb_ref[oi]
        eg = eg_ref[oi]
        eb = eb_ref[oi]
        mg = mg_ref[oi]
        mb = mb_ref[oi]

        for ch in range(_NCHUNK):
            sl = pl.ds(ch * _CB, _CB)
            h = h_ref[sl, si * C:(si + 1) * C]
            cm = convmr_ref[sl, :]
            ye = yext_ref[sl, :]
            c_out = jax.nn.relu(
                (cm[:, :C] - cmu[None, :]) * (cinv * cg)[None, :]
                + cb[None, :])
            y_ext = jnp.where((eg >= 0)[None, :], ye[:, :C], ye[:, C:])
            e_out = jax.nn.relu(
                (y_ext - emu[None, :]) * (einv * eg)[None, :] + eb[None, :])
            m_out = jax.nn.relu(
                (cm[:, C:] - mmu[None, :]) * (minv * mg)[None, :]
                + mb[None, :])
            o = (w_skip * h + w_conv * c_out + w_edge * e_out + w_mr * m_out)
            if oi == 0:
                if base_ref is not None:
                    o = base_ref[sl, :] + o
                out_ref[sl, :] = o
            else:
                out_ref[sl, :] = out_ref[sl, :] + o


def _opgroup(op_state, h_pack, g_pack, cw, cg, cb, ew, eg, eb, mw, mg, mb,
             weff, base=None):
    n_states = g_pack.shape[-1] // C
    has_base = base is not None
    body = functools.partial(_opgroup_body, tuple(op_state), n_states,
                             has_base)

    def wrapped(*refs):
        body(refs)

    n_in = 12 + (1 if has_base else 0)
    in_specs = [pl.BlockSpec(memory_space=pltpu.VMEM)] * (n_in - 1)
    in_specs.insert(11, pl.BlockSpec(memory_space=pltpu.SMEM))

    args = [h_pack, g_pack, cw, cg, cb, ew, eg, eb, mw, mg, mb, weff]
    if has_base:
        args.append(base)

    return pl.pallas_call(
        wrapped,
        out_shape=jax.ShapeDtypeStruct((R, C), jnp.float32),
        in_specs=in_specs,
        scratch_shapes=[
            pltpu.VMEM((R, 2 * C), jnp.float32),
            pltpu.VMEM((R, 2 * C), jnp.float32),
            pltpu.VMEM((R, n_states * C), jnp.float32),
        ],
    )(*args)


def kernel(s0, s1, weights, selected_idxs, pre0_w, pre0_g, pre0_b,
           pre1_w, pre1_g, pre1_b, conv_w, conv_g, conv_b,
           edge_w, edge_g, edge_b, mr_w, mr_g, mr_b):
    s0_rows = jnp.transpose(jnp.squeeze(s0, -1), (0, 2, 1)).reshape(R, C)
    s1_rows = jnp.transpose(jnp.squeeze(s1, -1), (0, 2, 1)).reshape(R, C)

    sel = jnp.asarray(selected_idxs, jnp.int32)
    onehot = (sel[:, None] == jnp.arange(5, dtype=jnp.int32)[None, :])
    weff = jnp.where((sel == -1)[:, None], weights,
                     onehot.astype(jnp.float32))
    weff = jnp.pad(weff, ((0, 0), (0, 3)))

    gidx, s0p, s1p = _k1(s0_rows, s1_rows, pre0_w, pre0_g, pre0_b,
                         pre1_w, pre1_g, pre1_b)
    gidx_flat = gidx.reshape(E)

    g_s0p = _sc_gather(s0p, gidx_flat).reshape(K, R, C)
    g_s1p = _sc_gather(s1p, gidx_flat).reshape(K, R, C)

    def params(idxs):
        ii = jnp.asarray(idxs, jnp.int32)
        return (conv_w[ii], conv_g[ii], conv_b[ii],
                edge_w[ii], edge_g[ii], edge_b[ii],
                mr_w[ii], mr_g[ii], mr_b[ii], weff[ii])

    h01 = jnp.concatenate([s0p, s1p], axis=1)
    g01 = jnp.concatenate([g_s0p, g_s1p], axis=-1)

    s2 = _opgroup((0, 1), h01, g01, *params([0, 1]))
    g_s2 = _sc_gather(s2, gidx_flat).reshape(K, R, C)

    part = _opgroup((0, 1), h01, g01, *params([2, 3]))
    s3 = _opgroup((0,), s2, g_s2, *params([4]), base=part)
    g_s3 = _sc_gather(s3, gidx_flat).reshape(K, R, C)

    part = _opgroup((0, 1), h01, g01, *params([5, 6]))
    h23 = jnp.concatenate([s2, s3], axis=1)
    g23 = jnp.concatenate([g_s2, g_s3], axis=-1)
    s4 = _opgroup((0, 1), h23, g23, *params([7, 8]), base=part)

    def to_ref_layout(rows):
        return jnp.transpose(rows.reshape(B, N, C), (0, 2, 1))[..., None]

    return jnp.concatenate(
        [to_ref_layout(s2), to_ref_layout(s3), to_ref_layout(s4)], axis=1)
